# Initial kernel scaffold; baseline (speedup 1.0000x reference)
import jax
import jax.numpy as jnp
from jax.experimental import pallas as pl


def kernel(x, pos_table):
    raise NotImplementedError("write your pallas kernel here")


# speedup vs baseline: 4.2271x; 4.2271x over previous
import functools

import jax
import jax.numpy as jnp
from jax.experimental import pallas as pl


def _body(x_ref, tmain_ref, t0_ref, o_ref):
    xb = x_ref[...]
    nz = jnp.any(xb != 0.0, axis=2)
    emb = jnp.where(nz[:, :, None], tmain_ref[...][None], t0_ref[...][None, :])
    o_ref[...] = xb + emb


@functools.partial(jax.jit, static_argnames=("interpret",))
def kernel(x, pos_table, interpret=False):
    B, L, D = x.shape
    BB = 64
    tmain = pos_table[1:]
    t0 = pos_table[0:1]
    grid = (B // BB,)
    return pl.pallas_call(
        _body,
        grid=grid,
        in_specs=[
            pl.BlockSpec((BB, L, D), lambda i: (i, 0, 0)),
            pl.BlockSpec((L, D), lambda i: (0, 0)),
            pl.BlockSpec((1, D), lambda i: (0, 0)),
        ],
        out_specs=pl.BlockSpec((BB, L, D), lambda i: (i, 0, 0)),
        out_shape=jax.ShapeDtypeStruct((B, L, D), x.dtype),
        interpret=interpret,
    )(x, tmain, t0)


# speedup vs baseline: 4.2557x; 1.0068x over previous
import functools

import jax
import jax.numpy as jnp
from jax.experimental import pallas as pl


def _body(x_ref, tmain_ref, t0_ref, o_ref):
    xb = x_ref[...]
    nz = jnp.any(xb != 0.0, axis=2)
    emb = jnp.where(nz[:, :, None], tmain_ref[...][None], t0_ref[...][None, :])
    o_ref[...] = xb + emb


@functools.partial(jax.jit, static_argnames=("interpret",))
def kernel(x, pos_table, interpret=False):
    B, L, D = x.shape
    BB = 128
    tmain = pos_table[1:]
    t0 = pos_table[0:1]
    grid = (B // BB,)
    return pl.pallas_call(
        _body,
        grid=grid,
        in_specs=[
            pl.BlockSpec((BB, L, D), lambda i: (i, 0, 0)),
            pl.BlockSpec((L, D), lambda i: (0, 0)),
            pl.BlockSpec((1, D), lambda i: (0, 0)),
        ],
        out_specs=pl.BlockSpec((BB, L, D), lambda i: (i, 0, 0)),
        out_shape=jax.ShapeDtypeStruct((B, L, D), x.dtype),
        interpret=interpret,
    )(x, tmain, t0)


# speedup vs baseline: 6.8282x; 1.6045x over previous
import functools

import jax
import jax.numpy as jnp
from jax.experimental import pallas as pl


def _body(x_ref, tmain_ref, t0_ref, s_ref, o_ref):
    xb = x_ref[...]
    bb = xb.shape[0]
    f = (xb != 0.0).astype(jnp.float32)
    cnt = jax.lax.dot_general(
        f.reshape(bb * 100, 128), s_ref[...],
        (((1,), (0,)), ((), ())),
        preferred_element_type=jnp.float32,
    ).reshape(bb, 100, 128)
    emb = jnp.where(cnt > 0.0, tmain_ref[...][None], t0_ref[...][None])
    o_ref[...] = xb + emb


@functools.partial(jax.jit, static_argnames=("interpret",))
def kernel(x, pos_table, interpret=False):
    B, L, D = x.shape
    BB = 128
    x2 = x.reshape(B, L // 2, 2 * D)
    tmain = pos_table[1:].reshape(L // 2, 2 * D)
    t0 = jnp.tile(pos_table[0], 2)[None, :]
    half = jnp.arange(2 * D, dtype=jnp.int32) // D
    s = (half[:, None] == half[None, :]).astype(jnp.float32)
    grid = (B // BB,)
    out = pl.pallas_call(
        _body,
        grid=grid,
        in_specs=[
            pl.BlockSpec((BB, L // 2, 2 * D), lambda i: (i, 0, 0)),
            pl.BlockSpec((L // 2, 2 * D), lambda i: (0, 0)),
            pl.BlockSpec((1, 2 * D), lambda i: (0, 0)),
            pl.BlockSpec((2 * D, 2 * D), lambda i: (0, 0)),
        ],
        out_specs=pl.BlockSpec((BB, L // 2, 2 * D), lambda i: (i, 0, 0)),
        out_shape=jax.ShapeDtypeStruct((B, L // 2, 2 * D), x.dtype),
        interpret=interpret,
    )(x2, tmain, t0, s)
    return out.reshape(B, L, D)
